# baseline (speedup 1.0000x reference)
import functools

import jax
import jax.numpy as jnp
from jax import lax
from jax.experimental import pallas as pl
from jax.experimental.pallas import tpu as pltpu
from jax.experimental.pallas import tpu_sc as plsc

N = 10000
E = 320000
DF = 128
HID = 128
GEN = 64
EMB = 128
P = 5
OUT = 40

NP = 10240
EP = 327680
NC, NS = 2, 16
NW = NC * NS
EPW = EP // NW
CH = 64
NCH = EPW // CH
NCHH = 40
SLOW = 1
KSLOW = 80
KFAST = 240
CBASE = 16 * KSLOW
RPT = NP // NS
CW = 128

BLK = 512
GRID = NP // BLK

_SC_MESH = dict(core_axis_name="c", subcore_axis_name="s")


@functools.partial(
    pl.kernel,
    out_type=jax.ShapeDtypeStruct((NC, NP, CW), jnp.float32),
    mesh=plsc.VectorSubcoreMesh(**_SC_MESH),
    scratch_types=[
        pltpu.VMEM((NCH, CH), jnp.int32),
        pltpu.VMEM((CH, CW), jnp.float32),
        pltpu.VMEM_SHARED((NP, CW), jnp.float32),
    ],
)
def _seg_cnt(dst2d, z16, o16, out_c, dst_v, ones_v, cacc):
    c = lax.axis_index("c")
    s = lax.axis_index("s")
    wid = s * NC + c

    pltpu.sync_copy(dst2d.at[pl.ds(wid * NCH, NCH)], dst_v)
    pltpu.sync_copy(z16, ones_v)
    for t in range(RPT // CH):
        pltpu.sync_copy(ones_v, cacc.at[pl.ds(s * RPT + t * CH, CH)])
    pltpu.sync_copy(o16, ones_v)
    plsc.subcore_barrier()

    def step(j, carry):
        pltpu.sync_copy(ones_v, cacc.at[dst_v.at[j]], add=True)
        return carry

    lax.fori_loop(0, NCH, step, 0)
    plsc.subcore_barrier()

    for t in range(RPT // CH):
        r0 = s * RPT + t * CH
        pltpu.sync_copy(cacc.at[pl.ds(r0, CH)], ones_v)
        pltpu.sync_copy(ones_v, out_c.at[c, pl.ds(r0, CH)])


NBUF = 4


def _seg_sum_body(table, src2d, dst2d, z128, out_p, src_v, dst_v,
                  r0_v, r1_v, r2_v, r3_v, gsems, acc):
    rows = [r0_v, r1_v, r2_v, r3_v]
    c = lax.axis_index("c")
    s = lax.axis_index("s")

    pltpu.sync_copy(z128, rows[0])
    for t in range(RPT // CH):
        pltpu.sync_copy(rows[0], acc.at[pl.ds(s * RPT + t * CH, CH)])
    plsc.subcore_barrier()

    kc = jnp.where(c == SLOW, KSLOW, KFAST)
    base = jnp.where(c == SLOW, s * KSLOW, CBASE + s * KFAST)

    def phase(h, carry):
        off = base + h * NCHH
        pltpu.sync_copy(src2d.at[pl.ds(off, NCHH)], src_v)
        pltpu.sync_copy(dst2d.at[pl.ds(off, NCHH)], dst_v)
        for b in range(NBUF):
            pltpu.async_copy(table.at[src_v.at[b]], rows[b], gsems.at[b])

        def step(i, carry2):
            j0 = i * NBUF
            for b in range(NBUF):
                j = j0 + b
                pltpu.make_async_copy(table.at[src_v.at[j]], rows[b],
                                      gsems.at[b]).wait()
                pltpu.sync_copy(rows[b], acc.at[dst_v.at[j]], add=True)
                pltpu.async_copy(table.at[src_v.at[j + NBUF]], rows[b],
                                 gsems.at[b])
            return carry2

        lax.fori_loop(0, NCHH // NBUF - 1, step, 0)
        for b in range(NBUF):
            j = NCHH - NBUF + b
            pltpu.make_async_copy(table.at[src_v.at[j]], rows[b],
                                  gsems.at[b]).wait()
            pltpu.sync_copy(rows[b], acc.at[dst_v.at[j]], add=True)
        return carry

    lax.fori_loop(0, kc // NCHH, phase, 0)
    plsc.subcore_barrier()

    for t in range(RPT // CH):
        r0 = s * RPT + t * CH
        pltpu.sync_copy(acc.at[pl.ds(r0, CH)], rows[0])
        pltpu.sync_copy(rows[0], out_p.at[c, pl.ds(r0, CH)])


def _make_seg_sum(w):
    @functools.partial(
        pl.kernel,
        out_type=jax.ShapeDtypeStruct((NC, NP, w), jnp.float32),
        mesh=plsc.VectorSubcoreMesh(**_SC_MESH),
        compiler_params=pltpu.CompilerParams(
            use_tc_tiling_on_sc=(w % 128 == 0)),
        scratch_types=[
            pltpu.VMEM((NCHH, CH), jnp.int32),
            pltpu.VMEM((NCHH, CH), jnp.int32),
            pltpu.VMEM((CH, w), jnp.float32),
            pltpu.VMEM((CH, w), jnp.float32),
            pltpu.VMEM((CH, w), jnp.float32),
            pltpu.VMEM((CH, w), jnp.float32),
            pltpu.SemaphoreType.DMA((NBUF,)),
            pltpu.VMEM_SHARED((NP, w), jnp.float32),
        ],
    )
    def k(*refs):
        _seg_sum_body(*refs)

    return k


_seg_sum = _make_seg_sum(DF)
_seg_sum64 = _make_seg_sum(GEN)


def _mean(p_ref, c_ref):
    cnt = jnp.maximum(c_ref[0] + c_ref[1], 1.0)
    return (p_ref[0] + p_ref[1]) / cnt


def _mm(a, w):
    return jnp.dot(a, w[...], preferred_element_type=jnp.float32)


def _stage_b_kernel(p_ref, c_ref, x_ref, wl, wr, b, w2l,
                    h_ref, mean_ref, hp_ref):
    mean = _mean(p_ref, c_ref)
    mean_ref[...] = mean
    h = jnp.maximum(_mm(mean, wl) + _mm(x_ref[...], wr) + b[...], 0.0)
    h_ref[...] = h
    hp_ref[...] = _mm(h, w2l)


def _stage_d_kernel(ph_ref, c_ref, h_ref, mx_ref, x_ref, nz_ref,
                    w2r, b2, wreg, breg, wf1, bf1, wf2, bf2, wfl, bfl,
                    wl0, bl0, wc0l, wc0r, bc0, wl1, bl1, wc1lp,
                    deg_ref, gen_ref, xf_ref, xe2_ref, xfp_ref):
    x_enc = _mean(ph_ref, c_ref) + _mm(h_ref[...], w2r) + b2[...]
    degree = jnp.maximum(_mm(x_enc, wreg) + breg[...], 0.0)
    deg_ref[...] = degree
    z = x_enc + nz_ref[...]
    g = jnp.maximum(_mm(z, wf1) + bf1[...], 0.0)
    g = jnp.maximum(_mm(g, wf2) + bf2[...], 0.0)
    gen = jnp.tanh(_mm(g, wfl) + bfl[...])
    gen_ref[...] = gen
    dcount = jnp.clip(jnp.round(degree), 0.0, float(P))
    x_emb = jnp.zeros((BLK, EMB), jnp.float32)
    for k in range(P):
        mk = jnp.where(dcount > float(k), 1.0, 0.0)
        x_emb = x_emb + mk * gen[:, k * EMB:(k + 1) * EMB]
    x_emb = x_emb * (1.0 / P)
    xe = _mm(x_emb, wl0) + bl0[...]
    c0 = _mm(mx_ref[...], wc0l) + _mm(x_ref[...], wc0r) + bc0[...]
    xf = jnp.maximum(c0 + xe, 0.0)
    xf_ref[...] = xf
    xe2_ref[...] = _mm(xe, wl1) + bl1[...]
    xfp_ref[...] = _mm(xf, wc1lp)


def _stage_f_kernel(pf_ref, c_ref, xf_ref, xe2_ref, wc1r, bc1, out_ref):
    mean_xfp = _mean(pf_ref, c_ref)
    out_ref[...] = (mean_xfp[:, :OUT] + _mm(xf_ref[...], wc1r)
                    + bc1[...] + xe2_ref[...])


def _row_spec(w):
    return pl.BlockSpec((BLK, w), lambda i: (i, 0))


def _part_spec(w):
    return pl.BlockSpec((NC, BLK, w), lambda i: (0, i, 0))


def _full_spec(shape):
    nd = len(shape)
    return pl.BlockSpec(shape, lambda i: (0,) * nd)


def kernel(x, edge_index, W1l, W1r, b1, W2l, W2r, b2, Wreg, breg, Wf1, bf1,
           Wf2, bf2, Wfl, bfl, Wc0l, Wc0r, bc0, WL0, bL0, Wc1l, Wc1r, bc1,
           WL1, bL1):
    f32 = jnp.float32
    x_p = jnp.pad(x, ((0, NP - N), (0, 0)))
    src = jnp.pad(edge_index[0].astype(jnp.int32), (0, EP - E))
    dst = jnp.pad(edge_index[1].astype(jnp.int32), (0, EP - E),
                  constant_values=N)
    src2d = src.reshape(EP // CH, CH)
    dst2d = dst.reshape(EP // CH, CH)
    z128 = jnp.zeros((CH, DF), f32)
    z64 = jnp.zeros((CH, GEN), f32)
    z16 = jnp.zeros((CH, CW), f32)
    o16 = jnp.ones((CH, CW), f32)

    noise = jax.random.normal(jax.random.key(42), (N, GEN), dtype=f32)
    noise_p = jnp.pad(noise, ((0, NP - N), (0, 0)))

    cnt_p = _seg_cnt(dst2d, z16, o16)[:, :, 0:1]
    p_x = _seg_sum(x_p, src2d, dst2d, z128)

    h, mean_x, hp = pl.pallas_call(
        _stage_b_kernel,
        grid=(GRID,),
        in_specs=[
            _part_spec(DF), _part_spec(1), _row_spec(DF),
            _full_spec((DF, HID)), _full_spec((DF, HID)), _full_spec((1, HID)),
            _full_spec((HID, GEN)),
        ],
        out_specs=(_row_spec(HID), _row_spec(DF), _row_spec(GEN)),
        out_shape=(jax.ShapeDtypeStruct((NP, HID), f32),
                   jax.ShapeDtypeStruct((NP, DF), f32),
                   jax.ShapeDtypeStruct((NP, GEN), f32)),
    )(p_x, cnt_p, x_p, W1l, W1r, b1.reshape(1, HID), W2l)

    p_h = _seg_sum64(hp, src2d, dst2d, z64)

    Wc1l_pad = jnp.pad(Wc1l, ((0, 0), (0, GEN - OUT)))
    deg, gen, xf, xe2, xfp = pl.pallas_call(
        _stage_d_kernel,
        grid=(GRID,),
        in_specs=[
            _part_spec(GEN), _part_spec(1), _row_spec(HID), _row_spec(DF),
            _row_spec(DF), _row_spec(GEN),
            _full_spec((HID, GEN)),
            _full_spec((1, GEN)),
            _full_spec((GEN, 1)), _full_spec((1, 1)),
            _full_spec((GEN, 256)), _full_spec((1, 256)),
            _full_spec((256, 2048)), _full_spec((1, 2048)),
            _full_spec((2048, P * EMB)), _full_spec((1, P * EMB)),
            _full_spec((EMB, HID)), _full_spec((1, HID)),
            _full_spec((DF, HID)), _full_spec((DF, HID)),
            _full_spec((1, HID)),
            _full_spec((EMB, OUT)), _full_spec((1, OUT)),
            _full_spec((HID, GEN)),
        ],
        out_specs=(_row_spec(1), _row_spec(P * EMB), _row_spec(HID),
                   _row_spec(OUT), _row_spec(GEN)),
        out_shape=(jax.ShapeDtypeStruct((NP, 1), f32),
                   jax.ShapeDtypeStruct((NP, P * EMB), f32),
                   jax.ShapeDtypeStruct((NP, HID), f32),
                   jax.ShapeDtypeStruct((NP, OUT), f32),
                   jax.ShapeDtypeStruct((NP, GEN), f32)),
    )(p_h, cnt_p, h, mean_x, x_p, noise_p,
      W2r, b2.reshape(1, GEN), Wreg, breg.reshape(1, 1),
      Wf1, bf1.reshape(1, 256), Wf2, bf2.reshape(1, 2048),
      Wfl, bfl.reshape(1, P * EMB), WL0, bL0.reshape(1, HID),
      Wc0l, Wc0r, bc0.reshape(1, HID), WL1, bL1.reshape(1, OUT), Wc1l_pad)

    p_xf = _seg_sum64(xfp, src2d, dst2d, z64)

    nc_pred = pl.pallas_call(
        _stage_f_kernel,
        grid=(GRID,),
        in_specs=[
            _part_spec(GEN), _part_spec(1), _row_spec(HID), _row_spec(OUT),
            _full_spec((HID, OUT)),
            _full_spec((1, OUT)),
        ],
        out_specs=_row_spec(OUT),
        out_shape=jax.ShapeDtypeStruct((NP, OUT), f32),
    )(p_xf, cnt_p, xf, xe2, Wc1r, bc1.reshape(1, OUT))

    return deg[:N], gen[:N], nc_pred[:N]


# speedup vs baseline: 1.0406x; 1.0406x over previous
import functools

import jax
import jax.numpy as jnp
from jax import lax
from jax.experimental import pallas as pl
from jax.experimental.pallas import tpu as pltpu
from jax.experimental.pallas import tpu_sc as plsc

N = 10000
E = 320000
DF = 128
HID = 128
GEN = 64
EMB = 128
P = 5
OUT = 40

NP = 10240
EP = 327680
NC, NS = 2, 16
NW = NC * NS
EPW = EP // NW
CH = 128
NCH = EPW // CH
NCHH = 16
SLOW = 0
KSLOW = 80
KFAST = 80
CBASE = 16 * KSLOW
RPT = NP // NS
CW = 128

BLK = 512
GRID = NP // BLK

_SC_MESH = dict(core_axis_name="c", subcore_axis_name="s")


@functools.partial(
    pl.kernel,
    out_type=jax.ShapeDtypeStruct((NC, NP, CW), jnp.float32),
    mesh=plsc.VectorSubcoreMesh(**_SC_MESH),
    scratch_types=[
        pltpu.VMEM((NCH, CH), jnp.int32),
        pltpu.VMEM((CH, CW), jnp.float32),
        pltpu.VMEM_SHARED((NP, CW), jnp.float32),
    ],
)
def _seg_cnt(dst2d, z16, o16, out_c, dst_v, ones_v, cacc):
    c = lax.axis_index("c")
    s = lax.axis_index("s")
    wid = s * NC + c

    pltpu.sync_copy(dst2d.at[pl.ds(wid * NCH, NCH)], dst_v)
    pltpu.sync_copy(z16, ones_v)
    for t in range(RPT // CH):
        pltpu.sync_copy(ones_v, cacc.at[pl.ds(s * RPT + t * CH, CH)])
    pltpu.sync_copy(o16, ones_v)
    plsc.subcore_barrier()

    def step(j, carry):
        pltpu.sync_copy(ones_v, cacc.at[dst_v.at[j]], add=True)
        return carry

    lax.fori_loop(0, NCH, step, 0)
    plsc.subcore_barrier()

    for t in range(RPT // CH):
        r0 = s * RPT + t * CH
        pltpu.sync_copy(cacc.at[pl.ds(r0, CH)], ones_v)
        pltpu.sync_copy(ones_v, out_c.at[c, pl.ds(r0, CH)])


NBUF = 2


def _seg_sum_body(table, src2d, dst2d, z128, out_p, src_v, dst_v,
                  r0_v, r1_v, gsems, acc):
    rows = [r0_v, r1_v]
    c = lax.axis_index("c")
    s = lax.axis_index("s")

    pltpu.sync_copy(z128, rows[0])
    for t in range(RPT // CH):
        pltpu.sync_copy(rows[0], acc.at[pl.ds(s * RPT + t * CH, CH)])
    plsc.subcore_barrier()

    kc = jnp.where(c == SLOW, KSLOW, KFAST)
    base = jnp.where(c == SLOW, s * KSLOW, CBASE + s * KFAST)

    def phase(h, carry):
        off = base + h * NCHH
        pltpu.sync_copy(src2d.at[pl.ds(off, NCHH)], src_v)
        pltpu.sync_copy(dst2d.at[pl.ds(off, NCHH)], dst_v)
        for b in range(NBUF):
            pltpu.async_copy(table.at[src_v.at[b]], rows[b], gsems.at[b])

        def step(i, carry2):
            j0 = i * NBUF
            for b in range(NBUF):
                j = j0 + b
                pltpu.make_async_copy(table.at[src_v.at[j]], rows[b],
                                      gsems.at[b]).wait()
                pltpu.sync_copy(rows[b], acc.at[dst_v.at[j]], add=True)
                pltpu.async_copy(table.at[src_v.at[j + NBUF]], rows[b],
                                 gsems.at[b])
            return carry2

        lax.fori_loop(0, NCHH // NBUF - 1, step, 0)
        for b in range(NBUF):
            j = NCHH - NBUF + b
            pltpu.make_async_copy(table.at[src_v.at[j]], rows[b],
                                  gsems.at[b]).wait()
            pltpu.sync_copy(rows[b], acc.at[dst_v.at[j]], add=True)
        return carry

    lax.fori_loop(0, kc // NCHH, phase, 0)
    plsc.subcore_barrier()

    for t in range(RPT // CH):
        r0 = s * RPT + t * CH
        pltpu.sync_copy(acc.at[pl.ds(r0, CH)], rows[0])
        pltpu.sync_copy(rows[0], out_p.at[c, pl.ds(r0, CH)])


def _make_seg_sum(w):
    @functools.partial(
        pl.kernel,
        out_type=jax.ShapeDtypeStruct((NC, NP, w), jnp.float32),
        mesh=plsc.VectorSubcoreMesh(**_SC_MESH),
        compiler_params=pltpu.CompilerParams(
            use_tc_tiling_on_sc=(w % 128 == 0)),
        scratch_types=[
            pltpu.VMEM((NCHH, CH), jnp.int32),
            pltpu.VMEM((NCHH, CH), jnp.int32),
            pltpu.VMEM((CH, w), jnp.float32),
            pltpu.VMEM((CH, w), jnp.float32),
            pltpu.SemaphoreType.DMA((NBUF,)),
            pltpu.VMEM_SHARED((NP, w), jnp.float32),
        ],
    )
    def k(*refs):
        _seg_sum_body(*refs)

    return k


_seg_sum = _make_seg_sum(DF)
_seg_sum64 = _make_seg_sum(GEN)


def _mean(p_ref, c_ref):
    cnt = jnp.maximum(c_ref[0] + c_ref[1], 1.0)
    return (p_ref[0] + p_ref[1]) / cnt


def _mm(a, w):
    return jnp.dot(a, w[...], preferred_element_type=jnp.float32)


def _stage_b_kernel(p_ref, c_ref, x_ref, wl, wr, b, w2l,
                    h_ref, mean_ref, hp_ref):
    mean = _mean(p_ref, c_ref)
    mean_ref[...] = mean
    h = jnp.maximum(_mm(mean, wl) + _mm(x_ref[...], wr) + b[...], 0.0)
    h_ref[...] = h
    hp_ref[...] = _mm(h, w2l)


def _stage_d_kernel(ph_ref, c_ref, h_ref, mx_ref, x_ref, nz_ref,
                    w2r, b2, wreg, breg, wf1, bf1, wf2, bf2, wfl, bfl,
                    wl0, bl0, wc0l, wc0r, bc0, wl1, bl1, wc1lp,
                    deg_ref, gen_ref, xf_ref, xe2_ref, xfp_ref):
    x_enc = _mean(ph_ref, c_ref) + _mm(h_ref[...], w2r) + b2[...]
    degree = jnp.maximum(_mm(x_enc, wreg) + breg[...], 0.0)
    deg_ref[...] = degree
    z = x_enc + nz_ref[...]
    g = jnp.maximum(_mm(z, wf1) + bf1[...], 0.0)
    g = jnp.maximum(_mm(g, wf2) + bf2[...], 0.0)
    gen = jnp.tanh(_mm(g, wfl) + bfl[...])
    gen_ref[...] = gen
    dcount = jnp.clip(jnp.round(degree), 0.0, float(P))
    x_emb = jnp.zeros((BLK, EMB), jnp.float32)
    for k in range(P):
        mk = jnp.where(dcount > float(k), 1.0, 0.0)
        x_emb = x_emb + mk * gen[:, k * EMB:(k + 1) * EMB]
    x_emb = x_emb * (1.0 / P)
    xe = _mm(x_emb, wl0) + bl0[...]
    c0 = _mm(mx_ref[...], wc0l) + _mm(x_ref[...], wc0r) + bc0[...]
    xf = jnp.maximum(c0 + xe, 0.0)
    xf_ref[...] = xf
    xe2_ref[...] = _mm(xe, wl1) + bl1[...]
    xfp_ref[...] = _mm(xf, wc1lp)


def _stage_f_kernel(pf_ref, c_ref, xf_ref, xe2_ref, wc1r, bc1, out_ref):
    mean_xfp = _mean(pf_ref, c_ref)
    out_ref[...] = (mean_xfp[:, :OUT] + _mm(xf_ref[...], wc1r)
                    + bc1[...] + xe2_ref[...])


def _row_spec(w):
    return pl.BlockSpec((BLK, w), lambda i: (i, 0))


def _part_spec(w):
    return pl.BlockSpec((NC, BLK, w), lambda i: (0, i, 0))


def _full_spec(shape):
    nd = len(shape)
    return pl.BlockSpec(shape, lambda i: (0,) * nd)


def kernel(x, edge_index, W1l, W1r, b1, W2l, W2r, b2, Wreg, breg, Wf1, bf1,
           Wf2, bf2, Wfl, bfl, Wc0l, Wc0r, bc0, WL0, bL0, Wc1l, Wc1r, bc1,
           WL1, bL1):
    f32 = jnp.float32
    x_p = jnp.pad(x, ((0, NP - N), (0, 0)))
    src = jnp.pad(edge_index[0].astype(jnp.int32), (0, EP - E))
    dst = jnp.pad(edge_index[1].astype(jnp.int32), (0, EP - E),
                  constant_values=N)
    src2d = src.reshape(EP // CH, CH)
    dst2d = dst.reshape(EP // CH, CH)
    z128 = jnp.zeros((CH, DF), f32)
    z64 = jnp.zeros((CH, GEN), f32)
    z16 = jnp.zeros((CH, CW), f32)
    o16 = jnp.ones((CH, CW), f32)

    noise = jax.random.normal(jax.random.key(42), (N, GEN), dtype=f32)
    noise_p = jnp.pad(noise, ((0, NP - N), (0, 0)))

    cnt_p = _seg_cnt(dst2d, z16, o16)[:, :, 0:1]
    p_x = _seg_sum(x_p, src2d, dst2d, z128)

    h, mean_x, hp = pl.pallas_call(
        _stage_b_kernel,
        grid=(GRID,),
        in_specs=[
            _part_spec(DF), _part_spec(1), _row_spec(DF),
            _full_spec((DF, HID)), _full_spec((DF, HID)), _full_spec((1, HID)),
            _full_spec((HID, GEN)),
        ],
        out_specs=(_row_spec(HID), _row_spec(DF), _row_spec(GEN)),
        out_shape=(jax.ShapeDtypeStruct((NP, HID), f32),
                   jax.ShapeDtypeStruct((NP, DF), f32),
                   jax.ShapeDtypeStruct((NP, GEN), f32)),
    )(p_x, cnt_p, x_p, W1l, W1r, b1.reshape(1, HID), W2l)

    p_h = _seg_sum64(hp, src2d, dst2d, z64)

    Wc1l_pad = jnp.pad(Wc1l, ((0, 0), (0, GEN - OUT)))
    deg, gen, xf, xe2, xfp = pl.pallas_call(
        _stage_d_kernel,
        grid=(GRID,),
        in_specs=[
            _part_spec(GEN), _part_spec(1), _row_spec(HID), _row_spec(DF),
            _row_spec(DF), _row_spec(GEN),
            _full_spec((HID, GEN)),
            _full_spec((1, GEN)),
            _full_spec((GEN, 1)), _full_spec((1, 1)),
            _full_spec((GEN, 256)), _full_spec((1, 256)),
            _full_spec((256, 2048)), _full_spec((1, 2048)),
            _full_spec((2048, P * EMB)), _full_spec((1, P * EMB)),
            _full_spec((EMB, HID)), _full_spec((1, HID)),
            _full_spec((DF, HID)), _full_spec((DF, HID)),
            _full_spec((1, HID)),
            _full_spec((EMB, OUT)), _full_spec((1, OUT)),
            _full_spec((HID, GEN)),
        ],
        out_specs=(_row_spec(1), _row_spec(P * EMB), _row_spec(HID),
                   _row_spec(OUT), _row_spec(GEN)),
        out_shape=(jax.ShapeDtypeStruct((NP, 1), f32),
                   jax.ShapeDtypeStruct((NP, P * EMB), f32),
                   jax.ShapeDtypeStruct((NP, HID), f32),
                   jax.ShapeDtypeStruct((NP, OUT), f32),
                   jax.ShapeDtypeStruct((NP, GEN), f32)),
    )(p_h, cnt_p, h, mean_x, x_p, noise_p,
      W2r, b2.reshape(1, GEN), Wreg, breg.reshape(1, 1),
      Wf1, bf1.reshape(1, 256), Wf2, bf2.reshape(1, 2048),
      Wfl, bfl.reshape(1, P * EMB), WL0, bL0.reshape(1, HID),
      Wc0l, Wc0r, bc0.reshape(1, HID), WL1, bL1.reshape(1, OUT), Wc1l_pad)

    p_xf = _seg_sum64(xfp, src2d, dst2d, z64)

    nc_pred = pl.pallas_call(
        _stage_f_kernel,
        grid=(GRID,),
        in_specs=[
            _part_spec(GEN), _part_spec(1), _row_spec(HID), _row_spec(OUT),
            _full_spec((HID, OUT)),
            _full_spec((1, OUT)),
        ],
        out_specs=_row_spec(OUT),
        out_shape=jax.ShapeDtypeStruct((NP, OUT), f32),
    )(p_xf, cnt_p, xf, xe2, Wc1r, bc1.reshape(1, OUT))

    return deg[:N], gen[:N], nc_pred[:N]


# speedup vs baseline: 1.0754x; 1.0335x over previous
import functools

import jax
import jax.numpy as jnp
from jax import lax
from jax.experimental import pallas as pl
from jax.experimental.pallas import tpu as pltpu
from jax.experimental.pallas import tpu_sc as plsc

N = 10000
E = 320000
DF = 128
HID = 128
GEN = 64
EMB = 128
P = 5
OUT = 40

NP = 10240
EP = 327680
NC, NS = 2, 16
NW = NC * NS
EPW = EP // NW
CH = 64
NCH = EPW // CH
NCHH = 40
SLOW = 0
KSLOW = 160
KFAST = 160
CBASE = 16 * KSLOW
RPT = NP // NS
CW = 128

BLK = 512
GRID = NP // BLK

_SC_MESH = dict(core_axis_name="c", subcore_axis_name="s")


@functools.partial(
    pl.kernel,
    out_type=jax.ShapeDtypeStruct((NC, NP, CW), jnp.float32),
    mesh=plsc.VectorSubcoreMesh(**_SC_MESH),
    scratch_types=[
        pltpu.VMEM((NCH, CH), jnp.int32),
        pltpu.VMEM((CH, CW), jnp.float32),
        pltpu.VMEM_SHARED((NP, CW), jnp.float32),
    ],
)
def _seg_cnt(dst2d, z16, o16, out_c, dst_v, ones_v, cacc):
    c = lax.axis_index("c")
    s = lax.axis_index("s")
    wid = s * NC + c

    pltpu.sync_copy(dst2d.at[pl.ds(wid * NCH, NCH)], dst_v)
    pltpu.sync_copy(z16, ones_v)
    for t in range(RPT // CH):
        pltpu.sync_copy(ones_v, cacc.at[pl.ds(s * RPT + t * CH, CH)])
    pltpu.sync_copy(o16, ones_v)
    plsc.subcore_barrier()

    def step(j, carry):
        pltpu.sync_copy(ones_v, cacc.at[dst_v.at[j]], add=True)
        return carry

    lax.fori_loop(0, NCH, step, 0)
    plsc.subcore_barrier()

    for t in range(RPT // CH):
        r0 = s * RPT + t * CH
        pltpu.sync_copy(cacc.at[pl.ds(r0, CH)], ones_v)
        pltpu.sync_copy(ones_v, out_c.at[c, pl.ds(r0, CH)])


NBUF = 4


def _seg_sum_body(table, src2d, dst2d, z128, out_p, src_v, dst_v,
                  r0_v, r1_v, r2_v, r3_v, gsems, acc):
    rows = [r0_v, r1_v, r2_v, r3_v]
    c = lax.axis_index("c")
    s = lax.axis_index("s")

    pltpu.sync_copy(z128, rows[0])
    for t in range(RPT // CH):
        pltpu.sync_copy(rows[0], acc.at[pl.ds(s * RPT + t * CH, CH)])
    plsc.subcore_barrier()

    kc = jnp.where(c == SLOW, KSLOW, KFAST)
    base = jnp.where(c == SLOW, s * KSLOW, CBASE + s * KFAST)

    def phase(h, carry):
        off = base + h * NCHH
        pltpu.sync_copy(src2d.at[pl.ds(off, NCHH)], src_v)
        pltpu.sync_copy(dst2d.at[pl.ds(off, NCHH)], dst_v)
        for b in range(NBUF):
            pltpu.async_copy(table.at[src_v.at[b]], rows[b], gsems.at[b])

        def step(i, carry2):
            j0 = i * NBUF
            for b in range(NBUF):
                j = j0 + b
                pltpu.make_async_copy(table.at[src_v.at[j]], rows[b],
                                      gsems.at[b]).wait()
                pltpu.sync_copy(rows[b], acc.at[dst_v.at[j]], add=True)
                pltpu.async_copy(table.at[src_v.at[j + NBUF]], rows[b],
                                 gsems.at[b])
            return carry2

        lax.fori_loop(0, NCHH // NBUF - 1, step, 0)
        for b in range(NBUF):
            j = NCHH - NBUF + b
            pltpu.make_async_copy(table.at[src_v.at[j]], rows[b],
                                  gsems.at[b]).wait()
            pltpu.sync_copy(rows[b], acc.at[dst_v.at[j]], add=True)
        return carry

    lax.fori_loop(0, kc // NCHH, phase, 0)
    plsc.subcore_barrier()

    for t in range(RPT // CH):
        r0 = s * RPT + t * CH
        pltpu.sync_copy(acc.at[pl.ds(r0, CH)], rows[0])
        pltpu.sync_copy(rows[0], out_p.at[c, pl.ds(r0, CH)])


def _make_seg_sum(w):
    @functools.partial(
        pl.kernel,
        out_type=jax.ShapeDtypeStruct((NC, NP, w), jnp.float32),
        mesh=plsc.VectorSubcoreMesh(**_SC_MESH),
        compiler_params=pltpu.CompilerParams(
            use_tc_tiling_on_sc=(w % 128 == 0)),
        scratch_types=[
            pltpu.VMEM((NCHH, CH), jnp.int32),
            pltpu.VMEM((NCHH, CH), jnp.int32),
            pltpu.VMEM((CH, w), jnp.float32),
            pltpu.VMEM((CH, w), jnp.float32),
            pltpu.VMEM((CH, w), jnp.float32),
            pltpu.VMEM((CH, w), jnp.float32),
            pltpu.SemaphoreType.DMA((NBUF,)),
            pltpu.VMEM_SHARED((NP, w), jnp.float32),
        ],
    )
    def k(*refs):
        _seg_sum_body(*refs)

    return k


_seg_sum = _make_seg_sum(DF)
_seg_sum64 = _make_seg_sum(GEN)


def _mean(p_ref, c_ref):
    cnt = jnp.maximum(c_ref[0] + c_ref[1], 1.0)
    return (p_ref[0] + p_ref[1]) / cnt


def _mm(a, w):
    return jnp.dot(a, w[...], preferred_element_type=jnp.float32)


def _stage_b_kernel(p_ref, c_ref, x_ref, wl, wr, b, w2l,
                    h_ref, mean_ref, hp_ref):
    mean = _mean(p_ref, c_ref)
    mean_ref[...] = mean
    h = jnp.maximum(_mm(mean, wl) + _mm(x_ref[...], wr) + b[...], 0.0)
    h_ref[...] = h
    hp_ref[...] = _mm(h, w2l)


def _stage_d_kernel(ph_ref, c_ref, h_ref, mx_ref, x_ref, nz_ref,
                    w2r, b2, wreg, breg, wf1, bf1, wf2, bf2, wfl, bfl,
                    wl0, bl0, wc0l, wc0r, bc0, wl1, bl1, wc1lp,
                    deg_ref, gen_ref, xf_ref, xe2_ref, xfp_ref):
    x_enc = _mean(ph_ref, c_ref) + _mm(h_ref[...], w2r) + b2[...]
    degree = jnp.maximum(_mm(x_enc, wreg) + breg[...], 0.0)
    deg_ref[...] = degree
    z = x_enc + nz_ref[...]
    g = jnp.maximum(_mm(z, wf1) + bf1[...], 0.0)
    g = jnp.maximum(_mm(g, wf2) + bf2[...], 0.0)
    gen = jnp.tanh(_mm(g, wfl) + bfl[...])
    gen_ref[...] = gen
    dcount = jnp.clip(jnp.round(degree), 0.0, float(P))
    x_emb = jnp.zeros((BLK, EMB), jnp.float32)
    for k in range(P):
        mk = jnp.where(dcount > float(k), 1.0, 0.0)
        x_emb = x_emb + mk * gen[:, k * EMB:(k + 1) * EMB]
    x_emb = x_emb * (1.0 / P)
    xe = _mm(x_emb, wl0) + bl0[...]
    c0 = _mm(mx_ref[...], wc0l) + _mm(x_ref[...], wc0r) + bc0[...]
    xf = jnp.maximum(c0 + xe, 0.0)
    xf_ref[...] = xf
    xe2_ref[...] = _mm(xe, wl1) + bl1[...]
    xfp_ref[...] = _mm(xf, wc1lp)


def _stage_f_kernel(pf_ref, c_ref, xf_ref, xe2_ref, wc1r, bc1, out_ref):
    mean_xfp = _mean(pf_ref, c_ref)
    out_ref[...] = (mean_xfp[:, :OUT] + _mm(xf_ref[...], wc1r)
                    + bc1[...] + xe2_ref[...])


def _row_spec(w):
    return pl.BlockSpec((BLK, w), lambda i: (i, 0))


def _part_spec(w):
    return pl.BlockSpec((NC, BLK, w), lambda i: (0, i, 0))


def _full_spec(shape):
    nd = len(shape)
    return pl.BlockSpec(shape, lambda i: (0,) * nd)


def kernel(x, edge_index, W1l, W1r, b1, W2l, W2r, b2, Wreg, breg, Wf1, bf1,
           Wf2, bf2, Wfl, bfl, Wc0l, Wc0r, bc0, WL0, bL0, Wc1l, Wc1r, bc1,
           WL1, bL1):
    f32 = jnp.float32
    x_p = jnp.pad(x, ((0, NP - N), (0, 0)))
    src = jnp.pad(edge_index[0].astype(jnp.int32), (0, EP - E))
    dst = jnp.pad(edge_index[1].astype(jnp.int32), (0, EP - E),
                  constant_values=N)
    src2d = src.reshape(EP // CH, CH)
    dst2d = dst.reshape(EP // CH, CH)
    z128 = jnp.zeros((CH, DF), f32)
    z64 = jnp.zeros((CH, GEN), f32)
    z16 = jnp.zeros((CH, CW), f32)
    o16 = jnp.ones((CH, CW), f32)

    noise = jax.random.normal(jax.random.key(42), (N, GEN), dtype=f32)
    noise_p = jnp.pad(noise, ((0, NP - N), (0, 0)))

    cnt_p = _seg_cnt(dst2d, z16, o16)[:, :, 0:1]
    p_x = _seg_sum(x_p, src2d, dst2d, z128)

    h, mean_x, hp = pl.pallas_call(
        _stage_b_kernel,
        grid=(GRID,),
        in_specs=[
            _part_spec(DF), _part_spec(1), _row_spec(DF),
            _full_spec((DF, HID)), _full_spec((DF, HID)), _full_spec((1, HID)),
            _full_spec((HID, GEN)),
        ],
        out_specs=(_row_spec(HID), _row_spec(DF), _row_spec(GEN)),
        out_shape=(jax.ShapeDtypeStruct((NP, HID), f32),
                   jax.ShapeDtypeStruct((NP, DF), f32),
                   jax.ShapeDtypeStruct((NP, GEN), f32)),
    )(p_x, cnt_p, x_p, W1l, W1r, b1.reshape(1, HID), W2l)

    p_h = _seg_sum64(hp, src2d, dst2d, z64)

    Wc1l_pad = jnp.pad(Wc1l, ((0, 0), (0, GEN - OUT)))
    deg, gen, xf, xe2, xfp = pl.pallas_call(
        _stage_d_kernel,
        grid=(GRID,),
        in_specs=[
            _part_spec(GEN), _part_spec(1), _row_spec(HID), _row_spec(DF),
            _row_spec(DF), _row_spec(GEN),
            _full_spec((HID, GEN)),
            _full_spec((1, GEN)),
            _full_spec((GEN, 1)), _full_spec((1, 1)),
            _full_spec((GEN, 256)), _full_spec((1, 256)),
            _full_spec((256, 2048)), _full_spec((1, 2048)),
            _full_spec((2048, P * EMB)), _full_spec((1, P * EMB)),
            _full_spec((EMB, HID)), _full_spec((1, HID)),
            _full_spec((DF, HID)), _full_spec((DF, HID)),
            _full_spec((1, HID)),
            _full_spec((EMB, OUT)), _full_spec((1, OUT)),
            _full_spec((HID, GEN)),
        ],
        out_specs=(_row_spec(1), _row_spec(P * EMB), _row_spec(HID),
                   _row_spec(OUT), _row_spec(GEN)),
        out_shape=(jax.ShapeDtypeStruct((NP, 1), f32),
                   jax.ShapeDtypeStruct((NP, P * EMB), f32),
                   jax.ShapeDtypeStruct((NP, HID), f32),
                   jax.ShapeDtypeStruct((NP, OUT), f32),
                   jax.ShapeDtypeStruct((NP, GEN), f32)),
    )(p_h, cnt_p, h, mean_x, x_p, noise_p,
      W2r, b2.reshape(1, GEN), Wreg, breg.reshape(1, 1),
      Wf1, bf1.reshape(1, 256), Wf2, bf2.reshape(1, 2048),
      Wfl, bfl.reshape(1, P * EMB), WL0, bL0.reshape(1, HID),
      Wc0l, Wc0r, bc0.reshape(1, HID), WL1, bL1.reshape(1, OUT), Wc1l_pad)

    p_xf = _seg_sum64(xfp, src2d, dst2d, z64)

    nc_pred = pl.pallas_call(
        _stage_f_kernel,
        grid=(GRID,),
        in_specs=[
            _part_spec(GEN), _part_spec(1), _row_spec(HID), _row_spec(OUT),
            _full_spec((HID, OUT)),
            _full_spec((1, OUT)),
        ],
        out_specs=_row_spec(OUT),
        out_shape=jax.ShapeDtypeStruct((NP, OUT), f32),
    )(p_xf, cnt_p, xf, xe2, Wc1r, bc1.reshape(1, OUT))

    return deg[:N], gen[:N], nc_pred[:N]


# speedup vs baseline: 1.0804x; 1.0046x over previous
import functools

import jax
import jax.numpy as jnp
from jax import lax
from jax.experimental import pallas as pl
from jax.experimental.pallas import tpu as pltpu
from jax.experimental.pallas import tpu_sc as plsc

N = 10000
E = 320000
DF = 128
HID = 128
GEN = 64
EMB = 128
P = 5
OUT = 40

NP = 10240
EP = 327680
NC, NS = 2, 16
NW = NC * NS
EPW = EP // NW
CH = 64
NCH = EPW // CH
NCHH = 40
SLOW = 0
KSLOW = 160
KFAST = 160
CBASE = 16 * KSLOW
RPT = NP // NS
CW = 64

BLK = 512
GRID = NP // BLK

_SC_MESH = dict(core_axis_name="c", subcore_axis_name="s")


@functools.partial(
    pl.kernel,
    out_type=jax.ShapeDtypeStruct((NC, NP, CW), jnp.float32),
    mesh=plsc.VectorSubcoreMesh(**_SC_MESH),
    compiler_params=pltpu.CompilerParams(use_tc_tiling_on_sc=False),
    scratch_types=[
        pltpu.VMEM((NCH, CH), jnp.int32),
        pltpu.VMEM((CH, CW), jnp.float32),
        pltpu.VMEM_SHARED((NP, CW), jnp.float32),
    ],
)
def _seg_cnt(dst2d, z16, o16, out_c, dst_v, ones_v, cacc):
    c = lax.axis_index("c")
    s = lax.axis_index("s")
    wid = s * NC + c

    pltpu.sync_copy(dst2d.at[pl.ds(wid * NCH, NCH)], dst_v)
    pltpu.sync_copy(z16, ones_v)
    for t in range(RPT // CH):
        pltpu.sync_copy(ones_v, cacc.at[pl.ds(s * RPT + t * CH, CH)])
    pltpu.sync_copy(o16, ones_v)
    plsc.subcore_barrier()

    def step(j, carry):
        pltpu.sync_copy(ones_v, cacc.at[dst_v.at[j]], add=True)
        return carry

    lax.fori_loop(0, NCH, step, 0)
    plsc.subcore_barrier()

    for t in range(RPT // CH):
        r0 = s * RPT + t * CH
        pltpu.sync_copy(cacc.at[pl.ds(r0, CH)], ones_v)
        pltpu.sync_copy(ones_v, out_c.at[c, pl.ds(r0, CH)])


NBUF = 4


def _seg_sum_body(table, src2d, dst2d, z128, out_p, src_v, dst_v,
                  r0_v, r1_v, r2_v, r3_v, gsems, acc):
    rows = [r0_v, r1_v, r2_v, r3_v]
    c = lax.axis_index("c")
    s = lax.axis_index("s")

    pltpu.sync_copy(z128, rows[0])
    for t in range(RPT // CH):
        pltpu.sync_copy(rows[0], acc.at[pl.ds(s * RPT + t * CH, CH)])
    plsc.subcore_barrier()

    kc = jnp.where(c == SLOW, KSLOW, KFAST)
    base = jnp.where(c == SLOW, s * KSLOW, CBASE + s * KFAST)

    def phase(h, carry):
        off = base + h * NCHH
        pltpu.sync_copy(src2d.at[pl.ds(off, NCHH)], src_v)
        pltpu.sync_copy(dst2d.at[pl.ds(off, NCHH)], dst_v)
        for b in range(NBUF):
            pltpu.async_copy(table.at[src_v.at[b]], rows[b], gsems.at[b])

        def step(i, carry2):
            j0 = i * NBUF
            for b in range(NBUF):
                j = j0 + b
                pltpu.make_async_copy(table.at[src_v.at[j]], rows[b],
                                      gsems.at[b]).wait()
                pltpu.sync_copy(rows[b], acc.at[dst_v.at[j]], add=True)
                pltpu.async_copy(table.at[src_v.at[j + NBUF]], rows[b],
                                 gsems.at[b])
            return carry2

        lax.fori_loop(0, NCHH // NBUF - 1, step, 0)
        for b in range(NBUF):
            j = NCHH - NBUF + b
            pltpu.make_async_copy(table.at[src_v.at[j]], rows[b],
                                  gsems.at[b]).wait()
            pltpu.sync_copy(rows[b], acc.at[dst_v.at[j]], add=True)
        return carry

    lax.fori_loop(0, kc // NCHH, phase, 0)
    plsc.subcore_barrier()

    for t in range(RPT // CH):
        r0 = s * RPT + t * CH
        pltpu.sync_copy(acc.at[pl.ds(r0, CH)], rows[0])
        pltpu.sync_copy(rows[0], out_p.at[c, pl.ds(r0, CH)])


def _make_seg_sum(w):
    @functools.partial(
        pl.kernel,
        out_type=jax.ShapeDtypeStruct((NC, NP, w), jnp.float32),
        mesh=plsc.VectorSubcoreMesh(**_SC_MESH),
        compiler_params=pltpu.CompilerParams(
            use_tc_tiling_on_sc=(w % 128 == 0)),
        scratch_types=[
            pltpu.VMEM((NCHH, CH), jnp.int32),
            pltpu.VMEM((NCHH, CH), jnp.int32),
            pltpu.VMEM((CH, w), jnp.float32),
            pltpu.VMEM((CH, w), jnp.float32),
            pltpu.VMEM((CH, w), jnp.float32),
            pltpu.VMEM((CH, w), jnp.float32),
            pltpu.SemaphoreType.DMA((NBUF,)),
            pltpu.VMEM_SHARED((NP, w), jnp.float32),
        ],
    )
    def k(*refs):
        _seg_sum_body(*refs)

    return k


_seg_sum = _make_seg_sum(DF)
_seg_sum64 = _make_seg_sum(GEN)


def _mean(p_ref, c_ref):
    cnt = jnp.maximum(c_ref[0] + c_ref[1], 1.0)
    return (p_ref[0] + p_ref[1]) / cnt


def _mm(a, w):
    return jnp.dot(a, w[...], preferred_element_type=jnp.float32)


def _stage_b_kernel(p_ref, c_ref, x_ref, wl, wr, b, w2l,
                    h_ref, mean_ref, hp_ref):
    mean = _mean(p_ref, c_ref)
    mean_ref[...] = mean
    h = jnp.maximum(_mm(mean, wl) + _mm(x_ref[...], wr) + b[...], 0.0)
    h_ref[...] = h
    hp_ref[...] = _mm(h, w2l)


def _stage_d_kernel(ph_ref, c_ref, h_ref, mx_ref, x_ref, nz_ref,
                    w2r, b2, wreg, breg, wf1, bf1, wf2, bf2, wfl, bfl,
                    wl0, bl0, wc0l, wc0r, bc0, wl1, bl1, wc1lp,
                    deg_ref, gen_ref, xf_ref, xe2_ref, xfp_ref):
    x_enc = _mean(ph_ref, c_ref) + _mm(h_ref[...], w2r) + b2[...]
    degree = jnp.maximum(_mm(x_enc, wreg) + breg[...], 0.0)
    deg_ref[...] = degree
    z = x_enc + nz_ref[...]
    g = jnp.maximum(_mm(z, wf1) + bf1[...], 0.0)
    g = jnp.maximum(_mm(g, wf2) + bf2[...], 0.0)
    gen = jnp.tanh(_mm(g, wfl) + bfl[...])
    gen_ref[...] = gen
    dcount = jnp.clip(jnp.round(degree), 0.0, float(P))
    x_emb = jnp.zeros((BLK, EMB), jnp.float32)
    for k in range(P):
        mk = jnp.where(dcount > float(k), 1.0, 0.0)
        x_emb = x_emb + mk * gen[:, k * EMB:(k + 1) * EMB]
    x_emb = x_emb * (1.0 / P)
    xe = _mm(x_emb, wl0) + bl0[...]
    c0 = _mm(mx_ref[...], wc0l) + _mm(x_ref[...], wc0r) + bc0[...]
    xf = jnp.maximum(c0 + xe, 0.0)
    xf_ref[...] = xf
    xe2_ref[...] = _mm(xe, wl1) + bl1[...]
    xfp_ref[...] = _mm(xf, wc1lp)


def _stage_f_kernel(pf_ref, c_ref, xf_ref, xe2_ref, wc1r, bc1, out_ref):
    mean_xfp = _mean(pf_ref, c_ref)
    out_ref[...] = (mean_xfp[:, :OUT] + _mm(xf_ref[...], wc1r)
                    + bc1[...] + xe2_ref[...])


def _row_spec(w):
    return pl.BlockSpec((BLK, w), lambda i: (i, 0))


def _part_spec(w):
    return pl.BlockSpec((NC, BLK, w), lambda i: (0, i, 0))


def _full_spec(shape):
    nd = len(shape)
    return pl.BlockSpec(shape, lambda i: (0,) * nd)


def kernel(x, edge_index, W1l, W1r, b1, W2l, W2r, b2, Wreg, breg, Wf1, bf1,
           Wf2, bf2, Wfl, bfl, Wc0l, Wc0r, bc0, WL0, bL0, Wc1l, Wc1r, bc1,
           WL1, bL1):
    f32 = jnp.float32
    x_p = jnp.pad(x, ((0, NP - N), (0, 0)))
    src = jnp.pad(edge_index[0].astype(jnp.int32), (0, EP - E))
    dst = jnp.pad(edge_index[1].astype(jnp.int32), (0, EP - E),
                  constant_values=N)
    src2d = src.reshape(EP // CH, CH)
    dst2d = dst.reshape(EP // CH, CH)
    z128 = jnp.zeros((CH, DF), f32)
    z64 = jnp.zeros((CH, GEN), f32)
    z16 = jnp.zeros((CH, CW), f32)
    o16 = jnp.ones((CH, CW), f32)

    noise = jax.random.normal(jax.random.key(42), (N, GEN), dtype=f32)
    noise_p = jnp.pad(noise, ((0, NP - N), (0, 0)))

    cnt_p = _seg_cnt(dst2d, z16, o16)[:, :, 0:1]
    p_x = _seg_sum(x_p, src2d, dst2d, z128)

    h, mean_x, hp = pl.pallas_call(
        _stage_b_kernel,
        grid=(GRID,),
        in_specs=[
            _part_spec(DF), _part_spec(1), _row_spec(DF),
            _full_spec((DF, HID)), _full_spec((DF, HID)), _full_spec((1, HID)),
            _full_spec((HID, GEN)),
        ],
        out_specs=(_row_spec(HID), _row_spec(DF), _row_spec(GEN)),
        out_shape=(jax.ShapeDtypeStruct((NP, HID), f32),
                   jax.ShapeDtypeStruct((NP, DF), f32),
                   jax.ShapeDtypeStruct((NP, GEN), f32)),
    )(p_x, cnt_p, x_p, W1l, W1r, b1.reshape(1, HID), W2l)

    p_h = _seg_sum64(hp, src2d, dst2d, z64)

    Wc1l_pad = jnp.pad(Wc1l, ((0, 0), (0, GEN - OUT)))
    deg, gen, xf, xe2, xfp = pl.pallas_call(
        _stage_d_kernel,
        grid=(GRID,),
        in_specs=[
            _part_spec(GEN), _part_spec(1), _row_spec(HID), _row_spec(DF),
            _row_spec(DF), _row_spec(GEN),
            _full_spec((HID, GEN)),
            _full_spec((1, GEN)),
            _full_spec((GEN, 1)), _full_spec((1, 1)),
            _full_spec((GEN, 256)), _full_spec((1, 256)),
            _full_spec((256, 2048)), _full_spec((1, 2048)),
            _full_spec((2048, P * EMB)), _full_spec((1, P * EMB)),
            _full_spec((EMB, HID)), _full_spec((1, HID)),
            _full_spec((DF, HID)), _full_spec((DF, HID)),
            _full_spec((1, HID)),
            _full_spec((EMB, OUT)), _full_spec((1, OUT)),
            _full_spec((HID, GEN)),
        ],
        out_specs=(_row_spec(1), _row_spec(P * EMB), _row_spec(HID),
                   _row_spec(OUT), _row_spec(GEN)),
        out_shape=(jax.ShapeDtypeStruct((NP, 1), f32),
                   jax.ShapeDtypeStruct((NP, P * EMB), f32),
                   jax.ShapeDtypeStruct((NP, HID), f32),
                   jax.ShapeDtypeStruct((NP, OUT), f32),
                   jax.ShapeDtypeStruct((NP, GEN), f32)),
    )(p_h, cnt_p, h, mean_x, x_p, noise_p,
      W2r, b2.reshape(1, GEN), Wreg, breg.reshape(1, 1),
      Wf1, bf1.reshape(1, 256), Wf2, bf2.reshape(1, 2048),
      Wfl, bfl.reshape(1, P * EMB), WL0, bL0.reshape(1, HID),
      Wc0l, Wc0r, bc0.reshape(1, HID), WL1, bL1.reshape(1, OUT), Wc1l_pad)

    p_xf = _seg_sum64(xfp, src2d, dst2d, z64)

    nc_pred = pl.pallas_call(
        _stage_f_kernel,
        grid=(GRID,),
        in_specs=[
            _part_spec(GEN), _part_spec(1), _row_spec(HID), _row_spec(OUT),
            _full_spec((HID, OUT)),
            _full_spec((1, OUT)),
        ],
        out_specs=_row_spec(OUT),
        out_shape=jax.ShapeDtypeStruct((NP, OUT), f32),
    )(p_xf, cnt_p, xf, xe2, Wc1r, bc1.reshape(1, OUT))

    return deg[:N], gen[:N], nc_pred[:N]
